# Initial kernel scaffold; baseline (speedup 1.0000x reference)
import functools

import jax
import jax.numpy as jnp
from jax import lax
from jax.experimental import pallas as pl
from jax.experimental.pallas import tpu as pltpu
from jax.experimental.pallas import tpu_sc as plsc

N = 10000
E = 320000
DIN = 128
H = 512
C = 16

NC = 2
NS = 16
EB = 128
RPAD = 2528
EP = RPAD * EB
BN = 1000
GN = N // BN
NROW = N // NS

_mesh = plsc.VectorSubcoreMesh(core_axis_name="c", subcore_axis_name="s",
                               num_cores=NC, num_subcores=NS)

_f32 = jnp.float32
_i32 = jnp.int32


def _make_deg_kernel():
    rpt = RPAD // (NC * NS)

    @functools.partial(
        pl.kernel,
        out_type=jax.ShapeDtypeStruct((NC * N,), _f32),
        mesh=_mesh,
        scratch_types=[
            pltpu.VMEM((rpt, EB), _i32),
            pltpu.VMEM((rpt, EB), _f32),
            pltpu.VMEM_SHARED((N,), _f32),
            pltpu.VMEM((1000,), _f32),
        ],
    )
    def deg_kernel(col_hbm, w_hbm, out_hbm, col_v, w_v, acc_sh, zbuf):
        cid = lax.axis_index("c")
        sid = lax.axis_index("s")
        jbase = (cid * NS + sid) * rpt
        pltpu.sync_copy(col_hbm.at[pl.ds(jbase, rpt)], col_v)
        pltpu.sync_copy(w_hbm.at[pl.ds(jbase, rpt)], w_v)

        @pl.when(sid == 0)
        def _():
            @pl.loop(0, 1000 // 16)
            def _(i):
                zbuf[pl.ds(i * 16, 16)] = jnp.zeros((16,), _f32)
            for z in range(N // 1000):
                pltpu.sync_copy(zbuf, acc_sh.at[pl.ds(z * 1000, 1000)])

        plsc.subcore_barrier()

        @pl.loop(0, rpt)
        def _(j):
            pltpu.sync_copy(w_v.at[j], acc_sh.at[col_v.at[j]], add=True)

        plsc.subcore_barrier()

        @pl.when(sid == 0)
        def _():
            pltpu.sync_copy(acc_sh, out_hbm.at[pl.ds(cid * N, N)])

    return deg_kernel


def _make_hop_kernel(ncT, Wc, cpsc, edge_split):
    rpt = RPAD // (NC * NS) if edge_split else RPAD // NS
    out_rows = 2 * N if edge_split else ncT * N
    zrows = 25

    scratch = [
        pltpu.VMEM((rpt, EB), _i32),
        pltpu.VMEM((rpt, EB), _i32),
        pltpu.VMEM((rpt, EB), _f32),
        pltpu.VMEM((EB, Wc), _f32),
        pltpu.VMEM_SHARED((N, Wc), _f32),
        pltpu.VMEM((zrows, Wc), _f32),
        pltpu.SemaphoreType.DMA,
    ]
    if ncT > 1:
        scratch.insert(1, pltpu.VMEM((rpt, EB), _i32))

    @functools.partial(
        pl.kernel,
        out_type=jax.ShapeDtypeStruct((out_rows, Wc), _f32),
        mesh=_mesh,
        scratch_types=scratch,
    )
    def hop_kernel(u_hbm, row_hbm, col_hbm, w_hbm, s_hbm, row_v, *rest):
        if ncT > 1:
            ridx_v, col_v, w_v, g, acc_sh, zbuf, sem = rest
        else:
            col_v, w_v, g, acc_sh, zbuf, sem = rest
            ridx_v = row_v
        cid = lax.axis_index("c")
        sid = lax.axis_index("s")
        jbase = ((cid * NS + sid) if edge_split else sid) * rpt
        pltpu.sync_copy(row_hbm.at[pl.ds(jbase, rpt)], row_v)
        pltpu.sync_copy(col_hbm.at[pl.ds(jbase, rpt)], col_v)
        pltpu.sync_copy(w_hbm.at[pl.ds(jbase, rpt)], w_v)

        for r in range(zrows):
            for f in range(Wc // 16):
                zbuf[r, pl.ds(f * 16, 16)] = jnp.zeros((16,), _f32)

        for ci in range(cpsc):
            if ncT > 1:
                chunk = cid * cpsc + ci
                off = chunk * N

                @pl.loop(0, rpt)
                def _(r):
                    offv = jnp.full((16,), off, _i32)
                    for f in range(EB // 16):
                        ridx_v[r, pl.ds(f * 16, 16)] = (
                            row_v[r, pl.ds(f * 16, 16)] + offv)
            else:
                off = 0

            for z in range(NROW // zrows):
                pltpu.sync_copy(zbuf,
                                acc_sh.at[pl.ds(sid * NROW + z * zrows, zrows)])
            plsc.subcore_barrier()

            @pl.loop(0, rpt)
            def _(j):
                pltpu.async_copy(u_hbm.at[ridx_v.at[j]], g, sem).wait()

                @pl.loop(0, EB)
                def _(e):
                    wb = plsc.load_gather(
                        w_v, [jnp.full((16,), j, _i32), jnp.full((16,), e, _i32)])
                    for f in range(Wc // 16):
                        v = g[e, pl.ds(f * 16, 16)]
                        g[e, pl.ds(f * 16, 16)] = v * wb

                pltpu.sync_copy(g, acc_sh.at[col_v.at[j]], add=True)

            plsc.subcore_barrier()
            base = (cid * N if edge_split else off) + sid * NROW
            pltpu.sync_copy(acc_sh.at[pl.ds(sid * NROW, NROW)],
                            s_hbm.at[pl.ds(base, NROW)])
            if ci + 1 < cpsc:
                plsc.subcore_barrier()

    return hop_kernel


_deg_call = _make_deg_kernel()
_hop_l1 = _make_hop_kernel(ncT=2, Wc=64, cpsc=1, edge_split=False)
_hop_l2 = _make_hop_kernel(ncT=4, Wc=128, cpsc=2, edge_split=False)
_hop_l3 = _make_hop_kernel(ncT=1, Wc=16, cpsc=1, edge_split=True)


def _t_dis(degA, degB):
    def body(a_ref, b_ref, dis_ref):
        d = a_ref[...] + b_ref[...]
        dis_ref[...] = jnp.where(d > 0, lax.rsqrt(jnp.where(d > 0, d, 1.0)), 0.0)

    return pl.pallas_call(
        body,
        grid=(GN,),
        in_specs=[pl.BlockSpec((BN, 1), lambda i: (i, 0)),
                  pl.BlockSpec((BN, 1), lambda i: (i, 0))],
        out_specs=pl.BlockSpec((BN, 1), lambda i: (i, 0)),
        out_shape=jax.ShapeDtypeStruct((N, 1), _f32),
    )(degA, degB)


def _t_l1start(x, dis, W0):
    def body(x_ref, dis_ref, w_ref, acc_ref, u_ref):
        c = pl.program_id(1)
        xb = x_ref[...]
        u_ref[...] = xb * dis_ref[...]
        part = jnp.dot(xb, w_ref[...], preferred_element_type=_f32)

        @pl.when(c == 0)
        def _():
            acc_ref[...] = part

        @pl.when(c != 0)
        def _():
            acc_ref[...] = acc_ref[...] + part

    return pl.pallas_call(
        body,
        grid=(GN, 2),
        in_specs=[pl.BlockSpec((BN, 64), lambda i, c: (i, c)),
                  pl.BlockSpec((BN, 1), lambda i, c: (i, 0)),
                  pl.BlockSpec((64, H), lambda i, c: (c, 0))],
        out_specs=[pl.BlockSpec((BN, H), lambda i, c: (i, 0)),
                   pl.BlockSpec((BN, 64), lambda i, c: (c * GN + i, 0))],
        out_shape=[jax.ShapeDtypeStruct((N, H), _f32),
                   jax.ShapeDtypeStruct((2 * N, 64), _f32)],
    )(x, dis, W0)


def _t_hopacc(s, dis, Wk, acc_in, ncT, Wc, last):
    def body(s_ref, dis_ref, w_ref, acc_in_ref, acc_ref, *maybe_u):
        c = pl.program_id(1)
        disb = dis_ref[...]
        hb = s_ref[...] * disb
        if maybe_u:
            maybe_u[0][...] = hb * disb
        part = jnp.dot(hb, w_ref[...], preferred_element_type=_f32)

        @pl.when(c == 0)
        def _():
            acc_ref[...] = acc_in_ref[...] + part

        @pl.when(c != 0)
        def _():
            acc_ref[...] = acc_ref[...] + part

    out_specs = [pl.BlockSpec((BN, H), lambda i, c: (i, 0))]
    out_shape = [jax.ShapeDtypeStruct((N, H), _f32)]
    if not last:
        out_specs.append(pl.BlockSpec((BN, Wc), lambda i, c: (c * GN + i, 0)))
        out_shape.append(jax.ShapeDtypeStruct((ncT * N, Wc), _f32))

    res = pl.pallas_call(
        body,
        grid=(GN, ncT),
        in_specs=[pl.BlockSpec((BN, Wc), lambda i, c: (c * GN + i, 0)),
                  pl.BlockSpec((BN, 1), lambda i, c: (i, 0)),
                  pl.BlockSpec((Wc, H), lambda i, c: (c, 0)),
                  pl.BlockSpec((BN, H), lambda i, c: (i, 0))],
        out_specs=out_specs,
        out_shape=out_shape,
    )(s, dis, Wk, acc_in)
    return res if not last else (res[0], None)


def _t_epi1(acc, b):
    def body(acc_ref, b_ref, y_ref, cs_ref):
        i = pl.program_id(0)
        t = acc_ref[...] + b_ref[...]
        y = jnp.where(t > 0, t, jnp.exp(jnp.minimum(t, 0.0)) - 1.0)
        y_ref[...] = y
        s = jnp.sum(y, axis=0, keepdims=True)

        @pl.when(i == 0)
        def _():
            cs_ref[...] = s

        @pl.when(i != 0)
        def _():
            cs_ref[...] = cs_ref[...] + s

    return pl.pallas_call(
        body,
        grid=(GN,),
        in_specs=[pl.BlockSpec((BN, H), lambda i: (i, 0)),
                  pl.BlockSpec((1, H), lambda i: (0, 0))],
        out_specs=[pl.BlockSpec((BN, H), lambda i: (i, 0)),
                   pl.BlockSpec((1, H), lambda i: (0, 0))],
        out_shape=[jax.ShapeDtypeStruct((N, H), _f32),
                   jax.ShapeDtypeStruct((1, H), _f32)],
    )(acc, b)


def _t_epi2(y, cs, ms):
    def body(y_ref, cs_ref, ms_ref, vs_ref):
        i = pl.program_id(0)
        mean = cs_ref[...] * (1.0 / N)
        d = y_ref[...] - ms_ref[...] * mean
        s = jnp.sum(d * d, axis=0, keepdims=True)

        @pl.when(i == 0)
        def _():
            vs_ref[...] = s

        @pl.when(i != 0)
        def _():
            vs_ref[...] = vs_ref[...] + s

    return pl.pallas_call(
        body,
        grid=(GN,),
        in_specs=[pl.BlockSpec((BN, H), lambda i: (i, 0)),
                  pl.BlockSpec((1, H), lambda i: (0, 0)),
                  pl.BlockSpec((1, H), lambda i: (0, 0))],
        out_specs=pl.BlockSpec((1, H), lambda i: (0, 0)),
        out_shape=jax.ShapeDtypeStruct((1, H), _f32),
    )(y, cs, ms)


def _gnorm_block(y, cs, vs, ms, nw, nb):
    mean = cs * (1.0 / N)
    d = y - ms * mean
    var = vs * (1.0 / N)
    return nw * d * lax.rsqrt(var + 1e-5) + nb


def _t_epi3(y, cs, vs, ms, nw, nb, dis, Wn0):
    def body(y_ref, cs_ref, vs_ref, ms_ref, nw_ref, nb_ref, dis_ref, w_ref,
             acc_ref, u_ref):
        c = pl.program_id(1)
        g = _gnorm_block(y_ref[...], cs_ref[...], vs_ref[...], ms_ref[...],
                         nw_ref[...], nb_ref[...])
        u_ref[...] = g * dis_ref[...]
        part = jnp.dot(g, w_ref[...], preferred_element_type=_f32)

        @pl.when(c == 0)
        def _():
            acc_ref[...] = part

        @pl.when(c != 0)
        def _():
            acc_ref[...] = acc_ref[...] + part

    stat = pl.BlockSpec((1, 128), lambda i, c: (0, c))
    return pl.pallas_call(
        body,
        grid=(GN, 4),
        in_specs=[pl.BlockSpec((BN, 128), lambda i, c: (i, c)),
                  stat, stat, stat, stat, stat,
                  pl.BlockSpec((BN, 1), lambda i, c: (i, 0)),
                  pl.BlockSpec((128, H), lambda i, c: (c, 0))],
        out_specs=[pl.BlockSpec((BN, H), lambda i, c: (i, 0)),
                   pl.BlockSpec((BN, 128), lambda i, c: (c * GN + i, 0))],
        out_shape=[jax.ShapeDtypeStruct((N, H), _f32),
                   jax.ShapeDtypeStruct((4 * N, 128), _f32)],
    )(y, cs, vs, ms, nw, nb, dis, Wn0)


def _t_epi3l3(y, cs, vs, ms, nw, nb, dis, W3):
    def body(y_ref, cs_ref, vs_ref, ms_ref, nw_ref, nb_ref, dis_ref, w_ref,
             p_ref, u_ref):
        c = pl.program_id(1)
        g = _gnorm_block(y_ref[...], cs_ref[...], vs_ref[...], ms_ref[...],
                         nw_ref[...], nb_ref[...])
        for k in range(4):
            part = jnp.dot(g, w_ref[k], preferred_element_type=_f32)

            @pl.when(c == 0)
            def _(part=part, k=k):
                p_ref[k] = part

            @pl.when(c != 0)
            def _(part=part, k=k):
                p_ref[k] = p_ref[k] + part

        u_ref[...] = p_ref[3] * dis_ref[...]

    stat = pl.BlockSpec((1, 128), lambda i, c: (0, c))
    return pl.pallas_call(
        body,
        grid=(GN, 4),
        in_specs=[pl.BlockSpec((BN, 128), lambda i, c: (i, c)),
                  stat, stat, stat, stat, stat,
                  pl.BlockSpec((BN, 1), lambda i, c: (i, 0)),
                  pl.BlockSpec((4, 128, C), lambda i, c: (0, c, 0))],
        out_specs=[pl.BlockSpec((4, BN, C), lambda i, c: (0, i, 0)),
                   pl.BlockSpec((BN, C), lambda i, c: (i, 0))],
        out_shape=[jax.ShapeDtypeStruct((4, N, C), _f32),
                   jax.ShapeDtypeStruct((N, C), _f32)],
    )(y, cs, vs, ms, nw, nb, dis, W3)


def _t_l3hop(s2, p, dis, k):
    def body(sa_ref, sb_ref, p_ref, dis_ref, u_ref):
        disb = dis_ref[...]
        q = disb * (sa_ref[...] + sb_ref[...]) + p_ref[0]
        u_ref[...] = disb * q

    return pl.pallas_call(
        body,
        grid=(GN,),
        in_specs=[pl.BlockSpec((BN, C), lambda i: (i, 0)),
                  pl.BlockSpec((BN, C), lambda i: (GN + i, 0)),
                  pl.BlockSpec((1, BN, C), lambda i: (3 - k, i, 0)),
                  pl.BlockSpec((BN, 1), lambda i: (i, 0))],
        out_specs=pl.BlockSpec((BN, C), lambda i: (i, 0)),
        out_shape=jax.ShapeDtypeStruct((N, C), _f32),
    )(s2, s2, p, dis)


def _t_l3final(s2, p, dis, b3):
    def body(sa_ref, sb_ref, p_ref, dis_ref, b_ref, out_ref):
        out_ref[...] = (dis_ref[...] * (sa_ref[...] + sb_ref[...])
                        + p_ref[0] + b_ref[...])

    return pl.pallas_call(
        body,
        grid=(GN,),
        in_specs=[pl.BlockSpec((BN, C), lambda i: (i, 0)),
                  pl.BlockSpec((BN, C), lambda i: (GN + i, 0)),
                  pl.BlockSpec((1, BN, C), lambda i: (0, i, 0)),
                  pl.BlockSpec((BN, 1), lambda i: (i, 0)),
                  pl.BlockSpec((1, C), lambda i: (0, 0))],
        out_specs=pl.BlockSpec((BN, C), lambda i: (i, 0)),
        out_shape=jax.ShapeDtypeStruct((N, C), _f32),
    )(s2, s2, p, dis, b3)


def kernel(x, weight, W1, b1, W2, b2, W3, b3, n1_w, n1_b, n1_ms, n2_w, n2_b,
           n2_ms, edge_index):
    row, col = edge_index[0], edge_index[1]
    padn = EP - E
    padidx = jnp.arange(padn, dtype=_i32) % N
    rowp = jnp.concatenate([row, padidx]).reshape(RPAD, EB)
    colp = jnp.concatenate([col, padidx]).reshape(RPAD, EB)
    wp = jnp.concatenate([weight, jnp.zeros((padn,), _f32)]).reshape(RPAD, EB)

    deg2 = _deg_call(colp, wp)
    dis = _t_dis(deg2[:N, None], deg2[N:, None])

    b1r, b2r, b3r = b1[None, :], b2[None, :], b3[None, :]
    ms1, ms2 = n1_ms[None, :], n2_ms[None, :]

    acc, u = _t_l1start(x, dis, W1[0])
    for k in range(1, 4):
        s = _hop_l1(u, rowp, colp, wp)
        acc, u = _t_hopacc(s, dis, W1[k], acc, ncT=2, Wc=64, last=(k == 3))
    y, cs = _t_epi1(acc, b1r)
    vs = _t_epi2(y, cs, ms1)
    acc, u = _t_epi3(y, cs, vs, ms1, n1_w[None, :], n1_b[None, :], dis, W2[0])

    for k in range(1, 4):
        s = _hop_l2(u, rowp, colp, wp)
        acc, u = _t_hopacc(s, dis, W2[k], acc, ncT=4, Wc=128, last=(k == 3))
    y, cs = _t_epi1(acc, b2r)
    vs = _t_epi2(y, cs, ms2)
    p, u = _t_epi3l3(y, cs, vs, ms2, n2_w[None, :], n2_b[None, :], dis, W3)

    out = None
    for k in range(1, 4):
        s2 = _hop_l3(u, rowp, colp, wp)
        if k < 3:
            u = _t_l3hop(s2, p, dis, k)
        else:
            out = _t_l3final(s2, p, dis, b3r)
    return out


# speedup vs baseline: 5.5069x; 5.5069x over previous
import functools

import jax
import jax.numpy as jnp
from jax import lax
from jax.experimental import pallas as pl
from jax.experimental.pallas import tpu as pltpu
from jax.experimental.pallas import tpu_sc as plsc

N = 10000
E = 320000
DIN = 128
H = 512
C = 16

NC = 2
NS = 16
EB = 128
RPAD = 2560
EP = RPAD * EB
BN = 1000
GN = N // BN
TROW = 640
LROW = N - 15 * TROW

_mesh = plsc.VectorSubcoreMesh(core_axis_name="c", subcore_axis_name="s",
                               num_cores=NC, num_subcores=NS)

_f32 = jnp.float32
_i32 = jnp.int32


def _make_deg_kernel():
    rpt = RPAD // (NC * NS)

    @functools.partial(
        pl.kernel,
        out_type=jax.ShapeDtypeStruct((NC * N,), _f32),
        mesh=_mesh,
        scratch_types=[
            pltpu.VMEM((rpt, EB), _i32),
            pltpu.VMEM((rpt, EB), _f32),
            pltpu.VMEM_SHARED((N,), _f32),
            pltpu.VMEM((1000,), _f32),
        ],
    )
    def deg_kernel(col_hbm, w_hbm, out_hbm, col_v, w_v, acc_sh, zbuf):
        cid = lax.axis_index("c")
        sid = lax.axis_index("s")
        jbase = (cid * NS + sid) * rpt
        pltpu.sync_copy(col_hbm.at[pl.ds(jbase, rpt)], col_v)
        pltpu.sync_copy(w_hbm.at[pl.ds(jbase, rpt)], w_v)

        @pl.when(sid == 0)
        def _():
            @pl.loop(0, 1000 // 16)
            def _(i):
                zbuf[pl.ds(i * 16, 16)] = jnp.zeros((16,), _f32)
            for z in range(N // 1000):
                pltpu.sync_copy(zbuf, acc_sh.at[pl.ds(z * 1000, 1000)])

        plsc.subcore_barrier()

        @pl.loop(0, rpt)
        def _(j):
            pltpu.sync_copy(w_v.at[j], acc_sh.at[col_v.at[j]], add=True)

        plsc.subcore_barrier()

        @pl.when(sid == 0)
        def _():
            for z in range(N // 1000):
                pltpu.sync_copy(acc_sh.at[pl.ds(z * 1000, 1000)], zbuf)
                pltpu.sync_copy(zbuf, out_hbm.at[pl.ds(cid * N + z * 1000, 1000)])

    return deg_kernel


def _make_hop_kernel(ncT, Wc, cpsc, edge_split):
    rpt = RPAD // (NC * NS) if edge_split else RPAD // NS
    out_rows = 2 * N if edge_split else ncT * N
    SB = 16
    ZR = 16
    OR = 80

    scratch = [
        pltpu.VMEM((SB, EB), _i32),
        pltpu.VMEM((SB, EB), _i32),
        pltpu.VMEM((SB * EB,), _f32),
        pltpu.VMEM((EB, Wc), _f32),
        pltpu.VMEM_SHARED((N, Wc), _f32),
        pltpu.VMEM((ZR, Wc), _f32),
        pltpu.SemaphoreType.DMA,
    ]

    @functools.partial(
        pl.kernel,
        out_type=jax.ShapeDtypeStruct((out_rows, Wc), _f32),
        mesh=_mesh,
        scratch_types=scratch,
        compiler_params=pltpu.CompilerParams(needs_layout_passes=False),
    )
    def hop_kernel(u_hbm, row_hbm, col_hbm, w_hbm, s_hbm, row_v, col_v, w_v, g,
                   acc_sh, zbuf, sem):
        cid = lax.axis_index("c")
        sid = lax.axis_index("s")
        jbase = ((cid * NS + sid) if edge_split else sid) * rpt

        for r in range(ZR):
            for f in range(Wc // 16):
                zbuf[r, pl.ds(f * 16, 16)] = jnp.zeros((16,), _f32)

        for ci in range(cpsc):
            if ncT > 1:
                off = (cid * cpsc + ci) * N
            else:
                off = 0

            @pl.when(sid < 15)
            def _():
                for z in range(TROW // ZR):
                    pltpu.sync_copy(
                        zbuf, acc_sh.at[pl.ds(sid * TROW + z * ZR, ZR)])

            @pl.when(sid == 15)
            def _():
                for z in range(LROW // ZR):
                    pltpu.sync_copy(
                        zbuf, acc_sh.at[pl.ds(15 * TROW + z * ZR, ZR)])

            plsc.subcore_barrier()

            @pl.loop(0, rpt // SB)
            def _(b):
                jb = jbase + b * SB
                pltpu.sync_copy(row_hbm.at[pl.ds(jb, SB)], row_v)
                pltpu.sync_copy(col_hbm.at[pl.ds(jb, SB)], col_v)
                pltpu.sync_copy(w_hbm.at[pl.ds(jb * EB, SB * EB)], w_v)
                if ncT > 1:
                    offv = jnp.full((16,), off, _i32)

                    @pl.loop(0, SB)
                    def _(r):
                        for f in range(EB // 16):
                            row_v[r, pl.ds(f * 16, 16)] = (
                                row_v[r, pl.ds(f * 16, 16)] + offv)

                @pl.loop(0, SB)
                def _(j):
                    pltpu.async_copy(u_hbm.at[row_v.at[j]], g, sem).wait()

                    @pl.loop(0, EB)
                    def _(e):
                        wb = plsc.load_gather(
                            w_v, [jnp.full((16,), j * EB + e, _i32)])
                        for f in range(Wc // 16):
                            v = g[e, pl.ds(f * 16, 16)]
                            g[e, pl.ds(f * 16, 16)] = v * wb

                    pltpu.sync_copy(g, acc_sh.at[col_v.at[j]], add=True)

            plsc.subcore_barrier()
            base = cid * N if edge_split else off
            gb = g.at[pl.ds(0, OR)]

            @pl.when(sid < 15)
            def _():
                for z in range(TROW // OR):
                    r0 = sid * TROW + z * OR
                    pltpu.sync_copy(acc_sh.at[pl.ds(r0, OR)], gb)
                    pltpu.sync_copy(gb, s_hbm.at[pl.ds(base + r0, OR)])

            @pl.when(sid == 15)
            def _():
                for z in range(LROW // OR):
                    r0 = 15 * TROW + z * OR
                    pltpu.sync_copy(acc_sh.at[pl.ds(r0, OR)], gb)
                    pltpu.sync_copy(gb, s_hbm.at[pl.ds(base + r0, OR)])

            if ci + 1 < cpsc:
                plsc.subcore_barrier()

    return hop_kernel


_deg_call = _make_deg_kernel()
_hop_es = _make_hop_kernel(ncT=1, Wc=128, cpsc=1, edge_split=True)
_hop_l2 = _make_hop_kernel(ncT=4, Wc=128, cpsc=2, edge_split=False)


def _t_dis(degA, degB):
    def body(a_ref, b_ref, dis_ref):
        d = a_ref[...] + b_ref[...]
        dis_ref[...] = jnp.where(d > 0, lax.rsqrt(jnp.where(d > 0, d, 1.0)), 0.0)

    return pl.pallas_call(
        body,
        grid=(GN,),
        in_specs=[pl.BlockSpec((BN, 1), lambda i: (i, 0)),
                  pl.BlockSpec((BN, 1), lambda i: (i, 0))],
        out_specs=pl.BlockSpec((BN, 1), lambda i: (i, 0)),
        out_shape=jax.ShapeDtypeStruct((N, 1), _f32),
    )(degA, degB)


def _t_l1start(x, dis, W0):
    def body(x_ref, dis_ref, w_ref, acc_ref, u_ref):
        xb = x_ref[...]
        u_ref[...] = xb * dis_ref[...]
        acc_ref[...] = jnp.dot(xb, w_ref[...], preferred_element_type=_f32)

    return pl.pallas_call(
        body,
        grid=(GN,),
        in_specs=[pl.BlockSpec((BN, DIN), lambda i: (i, 0)),
                  pl.BlockSpec((BN, 1), lambda i: (i, 0)),
                  pl.BlockSpec((DIN, H), lambda i: (0, 0))],
        out_specs=[pl.BlockSpec((BN, H), lambda i: (i, 0)),
                   pl.BlockSpec((BN, DIN), lambda i: (i, 0))],
        out_shape=[jax.ShapeDtypeStruct((N, H), _f32),
                   jax.ShapeDtypeStruct((N, DIN), _f32)],
    )(x, dis, W0)


def _t_hopacc_es(s2, dis, Wk, acc_in, last):
    def body(sa_ref, sb_ref, dis_ref, w_ref, acc_in_ref, acc_ref, *maybe_u):
        disb = dis_ref[...]
        hb = (sa_ref[...] + sb_ref[...]) * disb
        if maybe_u:
            maybe_u[0][...] = hb * disb
        acc_ref[...] = acc_in_ref[...] + jnp.dot(
            hb, w_ref[...], preferred_element_type=_f32)

    D = Wk.shape[0]
    out_specs = [pl.BlockSpec((BN, H), lambda i: (i, 0))]
    out_shape = [jax.ShapeDtypeStruct((N, H), _f32)]
    if not last:
        out_specs.append(pl.BlockSpec((BN, D), lambda i: (i, 0)))
        out_shape.append(jax.ShapeDtypeStruct((N, D), _f32))

    res = pl.pallas_call(
        body,
        grid=(GN,),
        in_specs=[pl.BlockSpec((BN, D), lambda i: (i, 0)),
                  pl.BlockSpec((BN, D), lambda i: (GN + i, 0)),
                  pl.BlockSpec((BN, 1), lambda i: (i, 0)),
                  pl.BlockSpec((D, H), lambda i: (0, 0)),
                  pl.BlockSpec((BN, H), lambda i: (i, 0))],
        out_specs=out_specs,
        out_shape=out_shape,
    )(s2, s2, dis, Wk, acc_in)
    return res if not last else (res[0], None)


def _t_hopacc(s, dis, Wk, acc_in, ncT, Wc, last):
    def body(s_ref, dis_ref, w_ref, acc_in_ref, acc_ref, *maybe_u):
        c = pl.program_id(1)
        disb = dis_ref[...]
        hb = s_ref[...] * disb
        if maybe_u:
            maybe_u[0][...] = hb * disb
        part = jnp.dot(hb, w_ref[...], preferred_element_type=_f32)

        @pl.when(c == 0)
        def _():
            acc_ref[...] = acc_in_ref[...] + part

        @pl.when(c != 0)
        def _():
            acc_ref[...] = acc_ref[...] + part

    out_specs = [pl.BlockSpec((BN, H), lambda i, c: (i, 0))]
    out_shape = [jax.ShapeDtypeStruct((N, H), _f32)]
    if not last:
        out_specs.append(pl.BlockSpec((BN, Wc), lambda i, c: (c * GN + i, 0)))
        out_shape.append(jax.ShapeDtypeStruct((ncT * N, Wc), _f32))

    res = pl.pallas_call(
        body,
        grid=(GN, ncT),
        in_specs=[pl.BlockSpec((BN, Wc), lambda i, c: (c * GN + i, 0)),
                  pl.BlockSpec((BN, 1), lambda i, c: (i, 0)),
                  pl.BlockSpec((Wc, H), lambda i, c: (c, 0)),
                  pl.BlockSpec((BN, H), lambda i, c: (i, 0))],
        out_specs=out_specs,
        out_shape=out_shape,
    )(s, dis, Wk, acc_in)
    return res if not last else (res[0], None)


def _t_epi1(acc, b):
    def body(acc_ref, b_ref, y_ref, cs_ref):
        i = pl.program_id(0)
        t = acc_ref[...] + b_ref[...]
        y = jnp.where(t > 0, t, jnp.exp(jnp.minimum(t, 0.0)) - 1.0)
        y_ref[...] = y
        s = jnp.sum(y, axis=0, keepdims=True)

        @pl.when(i == 0)
        def _():
            cs_ref[...] = s

        @pl.when(i != 0)
        def _():
            cs_ref[...] = cs_ref[...] + s

    return pl.pallas_call(
        body,
        grid=(GN,),
        in_specs=[pl.BlockSpec((BN, H), lambda i: (i, 0)),
                  pl.BlockSpec((1, H), lambda i: (0, 0))],
        out_specs=[pl.BlockSpec((BN, H), lambda i: (i, 0)),
                   pl.BlockSpec((1, H), lambda i: (0, 0))],
        out_shape=[jax.ShapeDtypeStruct((N, H), _f32),
                   jax.ShapeDtypeStruct((1, H), _f32)],
    )(acc, b)


def _t_epi2(y, cs, ms):
    def body(y_ref, cs_ref, ms_ref, vs_ref):
        i = pl.program_id(0)
        mean = cs_ref[...] * (1.0 / N)
        d = y_ref[...] - ms_ref[...] * mean
        s = jnp.sum(d * d, axis=0, keepdims=True)

        @pl.when(i == 0)
        def _():
            vs_ref[...] = s

        @pl.when(i != 0)
        def _():
            vs_ref[...] = vs_ref[...] + s

    return pl.pallas_call(
        body,
        grid=(GN,),
        in_specs=[pl.BlockSpec((BN, H), lambda i: (i, 0)),
                  pl.BlockSpec((1, H), lambda i: (0, 0)),
                  pl.BlockSpec((1, H), lambda i: (0, 0))],
        out_specs=pl.BlockSpec((1, H), lambda i: (0, 0)),
        out_shape=jax.ShapeDtypeStruct((1, H), _f32),
    )(y, cs, ms)


def _gnorm_block(y, cs, vs, ms, nw, nb):
    mean = cs * (1.0 / N)
    d = y - ms * mean
    var = vs * (1.0 / N)
    return nw * d * lax.rsqrt(var + 1e-5) + nb


def _t_epi3(y, cs, vs, ms, nw, nb, dis, Wn0):
    def body(y_ref, cs_ref, vs_ref, ms_ref, nw_ref, nb_ref, dis_ref, w_ref,
             acc_ref, u_ref):
        c = pl.program_id(1)
        g = _gnorm_block(y_ref[...], cs_ref[...], vs_ref[...], ms_ref[...],
                         nw_ref[...], nb_ref[...])
        u_ref[...] = g * dis_ref[...]
        part = jnp.dot(g, w_ref[...], preferred_element_type=_f32)

        @pl.when(c == 0)
        def _():
            acc_ref[...] = part

        @pl.when(c != 0)
        def _():
            acc_ref[...] = acc_ref[...] + part

    stat = pl.BlockSpec((1, 128), lambda i, c: (0, c))
    return pl.pallas_call(
        body,
        grid=(GN, 4),
        in_specs=[pl.BlockSpec((BN, 128), lambda i, c: (i, c)),
                  stat, stat, stat, stat, stat,
                  pl.BlockSpec((BN, 1), lambda i, c: (i, 0)),
                  pl.BlockSpec((128, H), lambda i, c: (c, 0))],
        out_specs=[pl.BlockSpec((BN, H), lambda i, c: (i, 0)),
                   pl.BlockSpec((BN, 128), lambda i, c: (c * GN + i, 0))],
        out_shape=[jax.ShapeDtypeStruct((N, H), _f32),
                   jax.ShapeDtypeStruct((4 * N, 128), _f32)],
    )(y, cs, vs, ms, nw, nb, dis, Wn0)


def _t_epi3l3(y, cs, vs, ms, nw, nb, dis, W3):
    def body(y_ref, cs_ref, vs_ref, ms_ref, nw_ref, nb_ref, dis_ref, w_ref,
             p_ref, u_ref):
        c = pl.program_id(1)
        g = _gnorm_block(y_ref[...], cs_ref[...], vs_ref[...], ms_ref[...],
                         nw_ref[...], nb_ref[...])
        for k in range(4):
            part = jnp.dot(g, w_ref[k], preferred_element_type=_f32)

            @pl.when(c == 0)
            def _(part=part, k=k):
                p_ref[k] = part

            @pl.when(c != 0)
            def _(part=part, k=k):
                p_ref[k] = p_ref[k] + part

        u_ref[...] = jnp.concatenate(
            [p_ref[3] * dis_ref[...], jnp.zeros((BN, 128 - C), _f32)], axis=1)

    stat = pl.BlockSpec((1, 128), lambda i, c: (0, c))
    return pl.pallas_call(
        body,
        grid=(GN, 4),
        in_specs=[pl.BlockSpec((BN, 128), lambda i, c: (i, c)),
                  stat, stat, stat, stat, stat,
                  pl.BlockSpec((BN, 1), lambda i, c: (i, 0)),
                  pl.BlockSpec((4, 128, C), lambda i, c: (0, c, 0))],
        out_specs=[pl.BlockSpec((4, BN, C), lambda i, c: (0, i, 0)),
                   pl.BlockSpec((BN, 128), lambda i, c: (i, 0))],
        out_shape=[jax.ShapeDtypeStruct((4, N, C), _f32),
                   jax.ShapeDtypeStruct((N, 128), _f32)],
    )(y, cs, vs, ms, nw, nb, dis, W3)


def _t_l3hop(s2, p, dis, k):
    def body(sa_ref, sb_ref, p_ref, dis_ref, u_ref):
        disb = dis_ref[...]
        q = disb * (sa_ref[...] + sb_ref[...])[:, :C] + p_ref[0]
        u_ref[...] = jnp.concatenate(
            [disb * q, jnp.zeros((BN, 128 - C), _f32)], axis=1)

    return pl.pallas_call(
        body,
        grid=(GN,),
        in_specs=[pl.BlockSpec((BN, 128), lambda i: (i, 0)),
                  pl.BlockSpec((BN, 128), lambda i: (GN + i, 0)),
                  pl.BlockSpec((1, BN, C), lambda i: (3 - k, i, 0)),
                  pl.BlockSpec((BN, 1), lambda i: (i, 0))],
        out_specs=pl.BlockSpec((BN, 128), lambda i: (i, 0)),
        out_shape=jax.ShapeDtypeStruct((N, 128), _f32),
    )(s2, s2, p, dis)


def _t_l3final(s2, p, dis, b3):
    def body(sa_ref, sb_ref, p_ref, dis_ref, b_ref, out_ref):
        out_ref[...] = (dis_ref[...] * (sa_ref[...] + sb_ref[...])[:, :C]
                        + p_ref[0] + b_ref[...])

    return pl.pallas_call(
        body,
        grid=(GN,),
        in_specs=[pl.BlockSpec((BN, 128), lambda i: (i, 0)),
                  pl.BlockSpec((BN, 128), lambda i: (GN + i, 0)),
                  pl.BlockSpec((1, BN, C), lambda i: (0, i, 0)),
                  pl.BlockSpec((BN, 1), lambda i: (i, 0)),
                  pl.BlockSpec((1, C), lambda i: (0, 0))],
        out_specs=pl.BlockSpec((BN, C), lambda i: (i, 0)),
        out_shape=jax.ShapeDtypeStruct((N, C), _f32),
    )(s2, s2, p, dis, b3)


def kernel(x, weight, W1, b1, W2, b2, W3, b3, n1_w, n1_b, n1_ms, n2_w, n2_b,
           n2_ms, edge_index):
    row, col = edge_index[0], edge_index[1]
    padn = EP - E
    padidx = jnp.arange(padn, dtype=_i32) % N
    rowp = jnp.concatenate([row, padidx]).reshape(RPAD, EB)
    colp = jnp.concatenate([col, padidx]).reshape(RPAD, EB)
    wp = jnp.concatenate([weight, jnp.zeros((padn,), _f32)]).reshape(RPAD, EB)

    wflat = wp.reshape(EP)
    deg2 = _deg_call(colp, wp)
    dis = _t_dis(deg2[:N, None], deg2[N:, None])

    b1r, b2r, b3r = b1[None, :], b2[None, :], b3[None, :]
    ms1, ms2 = n1_ms[None, :], n2_ms[None, :]

    acc, u = _t_l1start(x, dis, W1[0])
    for k in range(1, 4):
        s = _hop_es(u, rowp, colp, wflat)
        acc, u = _t_hopacc_es(s, dis, W1[k], acc, last=(k == 3))
    y, cs = _t_epi1(acc, b1r)
    vs = _t_epi2(y, cs, ms1)
    acc, u = _t_epi3(y, cs, vs, ms1, n1_w[None, :], n1_b[None, :], dis, W2[0])

    for k in range(1, 4):
        s = _hop_l2(u, rowp, colp, wflat)
        acc, u = _t_hopacc(s, dis, W2[k], acc, ncT=4, Wc=128, last=(k == 3))
    y, cs = _t_epi1(acc, b2r)
    vs = _t_epi2(y, cs, ms2)
    p, u = _t_epi3l3(y, cs, vs, ms2, n2_w[None, :], n2_b[None, :], dis, W3)

    out = None
    for k in range(1, 4):
        s2 = _hop_es(u, rowp, colp, wflat)
        if k < 3:
            u = _t_l3hop(s2, p, dis, k)
        else:
            out = _t_l3final(s2, p, dis, b3r)
    return out


# speedup vs baseline: 8.5778x; 1.5576x over previous
import functools

import jax
import jax.numpy as jnp
from jax import lax
from jax.experimental import pallas as pl
from jax.experimental.pallas import tpu as pltpu
from jax.experimental.pallas import tpu_sc as plsc

N = 10000
E = 320000
DIN = 128
H = 512
C = 16

NC = 2
NS = 16
EB = 128
RPAD = 2560
EP = RPAD * EB
BN = 1000
GN = N // BN
TROW = 640
LROW = N - 15 * TROW

_mesh = plsc.VectorSubcoreMesh(core_axis_name="c", subcore_axis_name="s",
                               num_cores=NC, num_subcores=NS)

_f32 = jnp.float32
_i32 = jnp.int32


def _make_deg_kernel():
    rpt = RPAD // (NC * NS)

    @functools.partial(
        pl.kernel,
        out_type=jax.ShapeDtypeStruct((NC * N,), _f32),
        mesh=_mesh,
        scratch_types=[
            pltpu.VMEM((rpt, EB), _i32),
            pltpu.VMEM((rpt, EB), _f32),
            pltpu.VMEM_SHARED((N,), _f32),
            pltpu.VMEM((1000,), _f32),
        ],
    )
    def deg_kernel(col_hbm, w_hbm, out_hbm, col_v, w_v, acc_sh, zbuf):
        cid = lax.axis_index("c")
        sid = lax.axis_index("s")
        jbase = (cid * NS + sid) * rpt
        pltpu.sync_copy(col_hbm.at[pl.ds(jbase, rpt)], col_v)
        pltpu.sync_copy(w_hbm.at[pl.ds(jbase, rpt)], w_v)

        @pl.when(sid == 0)
        def _():
            @pl.loop(0, 1000 // 16)
            def _(i):
                zbuf[pl.ds(i * 16, 16)] = jnp.zeros((16,), _f32)
            for z in range(N // 1000):
                pltpu.sync_copy(zbuf, acc_sh.at[pl.ds(z * 1000, 1000)])

        plsc.subcore_barrier()

        @pl.loop(0, rpt)
        def _(j):
            pltpu.sync_copy(w_v.at[j], acc_sh.at[col_v.at[j]], add=True)

        plsc.subcore_barrier()

        @pl.when(sid == 0)
        def _():
            for z in range(N // 1000):
                pltpu.sync_copy(acc_sh.at[pl.ds(z * 1000, 1000)], zbuf)
                pltpu.sync_copy(zbuf, out_hbm.at[pl.ds(cid * N + z * 1000, 1000)])

    return deg_kernel


def _make_hop_kernel(ncT, Wc, cpsc, edge_split, scale_cols=None):
    rpt = RPAD // (NC * NS) if edge_split else RPAD // NS
    out_rows = 2 * N if edge_split else ncT * N
    SB = 16
    ZR = 16
    OR = 80
    UE = 4
    if scale_cols is None:
        scale_cols = Wc

    scratch = [
        pltpu.VMEM((SB, EB), _i32),
        pltpu.VMEM((SB, EB), _i32),
        pltpu.VMEM((SB * EB,), _f32),
        pltpu.VMEM((EB, Wc), _f32),
        pltpu.VMEM((EB, Wc), _f32),
        pltpu.VMEM_SHARED((N, Wc), _f32),
        pltpu.VMEM((ZR, Wc), _f32),
        pltpu.SemaphoreType.DMA,
        pltpu.SemaphoreType.DMA,
        pltpu.SemaphoreType.DMA,
        pltpu.SemaphoreType.DMA,
    ]

    @functools.partial(
        pl.kernel,
        out_type=jax.ShapeDtypeStruct((out_rows, Wc), _f32),
        mesh=_mesh,
        scratch_types=scratch,
        compiler_params=pltpu.CompilerParams(needs_layout_passes=False),
    )
    def hop_kernel(u_hbm, row_hbm, col_hbm, w_hbm, s_hbm, row_v, col_v, w_v,
                   g0, g1, acc_sh, zbuf, sg0, sg1, ss0, ss1):
        cid = lax.axis_index("c")
        sid = lax.axis_index("s")
        jbase = ((cid * NS + sid) if edge_split else sid) * rpt
        gs = [(g0, sg0, ss0), (g1, sg1, ss1)]

        for r in range(ZR):
            for f in range(Wc // 16):
                zbuf[r, pl.ds(f * 16, 16)] = jnp.zeros((16,), _f32)

        def scale(g, j):
            @pl.loop(0, EB // UE)
            def _(eg):
                e0 = eg * UE
                for q in range(UE):
                    e = e0 + q
                    wb = plsc.load_gather(
                        w_v, [jnp.full((16,), j * EB + e, _i32)])
                    for f in range(scale_cols // 16):
                        v = g[e, pl.ds(f * 16, 16)]
                        g[e, pl.ds(f * 16, 16)] = v * wb

        for ci in range(cpsc):
            if ncT > 1:
                off = (cid * cpsc + ci) * N
            else:
                off = 0

            @pl.when(sid < 15)
            def _():
                for z in range(TROW // ZR):
                    pltpu.sync_copy(
                        zbuf, acc_sh.at[pl.ds(sid * TROW + z * ZR, ZR)])

            @pl.when(sid == 15)
            def _():
                for z in range(LROW // ZR):
                    pltpu.sync_copy(
                        zbuf, acc_sh.at[pl.ds(15 * TROW + z * ZR, ZR)])

            plsc.subcore_barrier()

            @pl.loop(0, rpt // SB)
            def _(b):
                jb = jbase + b * SB
                pltpu.sync_copy(row_hbm.at[pl.ds(jb, SB)], row_v)
                pltpu.sync_copy(col_hbm.at[pl.ds(jb, SB)], col_v)
                pltpu.sync_copy(w_hbm.at[pl.ds(jb * EB, SB * EB)], w_v)
                if ncT > 1:
                    offv = jnp.full((16,), off, _i32)

                    @pl.loop(0, SB)
                    def _(r):
                        for f in range(EB // 16):
                            row_v[r, pl.ds(f * 16, 16)] = (
                                row_v[r, pl.ds(f * 16, 16)] + offv)

                gdesc = [None, None]
                sdesc = [None, None]
                gdesc[0] = pltpu.async_copy(u_hbm.at[row_v.at[0]], g0, sg0)
                for j in range(SB):
                    cg, csg, css = gs[j % 2]
                    if j + 1 < SB:
                        ng, nsg, _ = gs[(j + 1) % 2]
                        if sdesc[(j + 1) % 2] is not None:
                            sdesc[(j + 1) % 2].wait()
                        gdesc[(j + 1) % 2] = pltpu.async_copy(
                            u_hbm.at[row_v.at[j + 1]], ng, nsg)
                    gdesc[j % 2].wait()
                    scale(cg, j)
                    sdesc[j % 2] = pltpu.async_copy(
                        cg, acc_sh.at[col_v.at[j]], css, add=True)
                sdesc[0].wait()
                sdesc[1].wait()

            plsc.subcore_barrier()
            base = cid * N if edge_split else off
            gb = g0.at[pl.ds(0, OR)]

            @pl.when(sid < 15)
            def _():
                for z in range(TROW // OR):
                    r0 = sid * TROW + z * OR
                    pltpu.sync_copy(acc_sh.at[pl.ds(r0, OR)], gb)
                    pltpu.sync_copy(gb, s_hbm.at[pl.ds(base + r0, OR)])

            @pl.when(sid == 15)
            def _():
                for z in range(LROW // OR):
                    r0 = 15 * TROW + z * OR
                    pltpu.sync_copy(acc_sh.at[pl.ds(r0, OR)], gb)
                    pltpu.sync_copy(gb, s_hbm.at[pl.ds(base + r0, OR)])

            if ci + 1 < cpsc:
                plsc.subcore_barrier()

    return hop_kernel


_deg_call = _make_deg_kernel()
_hop_es = _make_hop_kernel(ncT=1, Wc=128, cpsc=1, edge_split=True)
_hop_l2 = _make_hop_kernel(ncT=4, Wc=128, cpsc=2, edge_split=False)
_hop_es16 = _make_hop_kernel(ncT=1, Wc=128, cpsc=1, edge_split=True,
                             scale_cols=16)


def _t_dis(degA, degB):
    def body(a_ref, b_ref, dis_ref):
        d = a_ref[...] + b_ref[...]
        dis_ref[...] = jnp.where(d > 0, lax.rsqrt(jnp.where(d > 0, d, 1.0)), 0.0)

    return pl.pallas_call(
        body,
        grid=(GN,),
        in_specs=[pl.BlockSpec((BN, 1), lambda i: (i, 0)),
                  pl.BlockSpec((BN, 1), lambda i: (i, 0))],
        out_specs=pl.BlockSpec((BN, 1), lambda i: (i, 0)),
        out_shape=jax.ShapeDtypeStruct((N, 1), _f32),
    )(degA, degB)


def _t_l1start(x, dis, W0):
    def body(x_ref, dis_ref, w_ref, acc_ref, u_ref):
        xb = x_ref[...]
        u_ref[...] = xb * dis_ref[...]
        acc_ref[...] = jnp.dot(xb, w_ref[...], preferred_element_type=_f32)

    return pl.pallas_call(
        body,
        grid=(GN,),
        in_specs=[pl.BlockSpec((BN, DIN), lambda i: (i, 0)),
                  pl.BlockSpec((BN, 1), lambda i: (i, 0)),
                  pl.BlockSpec((DIN, H), lambda i: (0, 0))],
        out_specs=[pl.BlockSpec((BN, H), lambda i: (i, 0)),
                   pl.BlockSpec((BN, DIN), lambda i: (i, 0))],
        out_shape=[jax.ShapeDtypeStruct((N, H), _f32),
                   jax.ShapeDtypeStruct((N, DIN), _f32)],
    )(x, dis, W0)


def _t_hopacc_es(s2, dis, Wk, acc_in, last):
    def body(sa_ref, sb_ref, dis_ref, w_ref, acc_in_ref, acc_ref, *maybe_u):
        disb = dis_ref[...]
        hb = (sa_ref[...] + sb_ref[...]) * disb
        if maybe_u:
            maybe_u[0][...] = hb * disb
        acc_ref[...] = acc_in_ref[...] + jnp.dot(
            hb, w_ref[...], preferred_element_type=_f32)

    D = Wk.shape[0]
    out_specs = [pl.BlockSpec((BN, H), lambda i: (i, 0))]
    out_shape = [jax.ShapeDtypeStruct((N, H), _f32)]
    if not last:
        out_specs.append(pl.BlockSpec((BN, D), lambda i: (i, 0)))
        out_shape.append(jax.ShapeDtypeStruct((N, D), _f32))

    res = pl.pallas_call(
        body,
        grid=(GN,),
        in_specs=[pl.BlockSpec((BN, D), lambda i: (i, 0)),
                  pl.BlockSpec((BN, D), lambda i: (GN + i, 0)),
                  pl.BlockSpec((BN, 1), lambda i: (i, 0)),
                  pl.BlockSpec((D, H), lambda i: (0, 0)),
                  pl.BlockSpec((BN, H), lambda i: (i, 0))],
        out_specs=out_specs,
        out_shape=out_shape,
    )(s2, s2, dis, Wk, acc_in)
    return res if not last else (res[0], None)


def _t_hopacc(s, dis, Wk, acc_in, ncT, Wc, last):
    def body(s_ref, dis_ref, w_ref, acc_in_ref, acc_ref, *maybe_u):
        c = pl.program_id(1)
        disb = dis_ref[...]
        hb = s_ref[...] * disb
        if maybe_u:
            maybe_u[0][...] = hb * disb
        part = jnp.dot(hb, w_ref[...], preferred_element_type=_f32)

        @pl.when(c == 0)
        def _():
            acc_ref[...] = acc_in_ref[...] + part

        @pl.when(c != 0)
        def _():
            acc_ref[...] = acc_ref[...] + part

    out_specs = [pl.BlockSpec((BN, H), lambda i, c: (i, 0))]
    out_shape = [jax.ShapeDtypeStruct((N, H), _f32)]
    if not last:
        out_specs.append(pl.BlockSpec((BN, Wc), lambda i, c: (c * GN + i, 0)))
        out_shape.append(jax.ShapeDtypeStruct((ncT * N, Wc), _f32))

    res = pl.pallas_call(
        body,
        grid=(GN, ncT),
        in_specs=[pl.BlockSpec((BN, Wc), lambda i, c: (c * GN + i, 0)),
                  pl.BlockSpec((BN, 1), lambda i, c: (i, 0)),
                  pl.BlockSpec((Wc, H), lambda i, c: (c, 0)),
                  pl.BlockSpec((BN, H), lambda i, c: (i, 0))],
        out_specs=out_specs,
        out_shape=out_shape,
    )(s, dis, Wk, acc_in)
    return res if not last else (res[0], None)


def _t_epi1(acc, b):
    def body(acc_ref, b_ref, y_ref, cs_ref):
        i = pl.program_id(0)
        t = acc_ref[...] + b_ref[...]
        y = jnp.where(t > 0, t, jnp.exp(jnp.minimum(t, 0.0)) - 1.0)
        y_ref[...] = y
        s = jnp.sum(y, axis=0, keepdims=True)

        @pl.when(i == 0)
        def _():
            cs_ref[...] = s

        @pl.when(i != 0)
        def _():
            cs_ref[...] = cs_ref[...] + s

    return pl.pallas_call(
        body,
        grid=(GN,),
        in_specs=[pl.BlockSpec((BN, H), lambda i: (i, 0)),
                  pl.BlockSpec((1, H), lambda i: (0, 0))],
        out_specs=[pl.BlockSpec((BN, H), lambda i: (i, 0)),
                   pl.BlockSpec((1, H), lambda i: (0, 0))],
        out_shape=[jax.ShapeDtypeStruct((N, H), _f32),
                   jax.ShapeDtypeStruct((1, H), _f32)],
    )(acc, b)


def _t_epi2(y, cs, ms):
    def body(y_ref, cs_ref, ms_ref, vs_ref):
        i = pl.program_id(0)
        mean = cs_ref[...] * (1.0 / N)
        d = y_ref[...] - ms_ref[...] * mean
        s = jnp.sum(d * d, axis=0, keepdims=True)

        @pl.when(i == 0)
        def _():
            vs_ref[...] = s

        @pl.when(i != 0)
        def _():
            vs_ref[...] = vs_ref[...] + s

    return pl.pallas_call(
        body,
        grid=(GN,),
        in_specs=[pl.BlockSpec((BN, H), lambda i: (i, 0)),
                  pl.BlockSpec((1, H), lambda i: (0, 0)),
                  pl.BlockSpec((1, H), lambda i: (0, 0))],
        out_specs=pl.BlockSpec((1, H), lambda i: (0, 0)),
        out_shape=jax.ShapeDtypeStruct((1, H), _f32),
    )(y, cs, ms)


def _gnorm_block(y, cs, vs, ms, nw, nb):
    mean = cs * (1.0 / N)
    d = y - ms * mean
    var = vs * (1.0 / N)
    return nw * d * lax.rsqrt(var + 1e-5) + nb


def _t_epi3(y, cs, vs, ms, nw, nb, dis, Wn0):
    def body(y_ref, cs_ref, vs_ref, ms_ref, nw_ref, nb_ref, dis_ref, w_ref,
             acc_ref, u_ref):
        c = pl.program_id(1)
        g = _gnorm_block(y_ref[...], cs_ref[...], vs_ref[...], ms_ref[...],
                         nw_ref[...], nb_ref[...])
        u_ref[...] = g * dis_ref[...]
        part = jnp.dot(g, w_ref[...], preferred_element_type=_f32)

        @pl.when(c == 0)
        def _():
            acc_ref[...] = part

        @pl.when(c != 0)
        def _():
            acc_ref[...] = acc_ref[...] + part

    stat = pl.BlockSpec((1, 128), lambda i, c: (0, c))
    return pl.pallas_call(
        body,
        grid=(GN, 4),
        in_specs=[pl.BlockSpec((BN, 128), lambda i, c: (i, c)),
                  stat, stat, stat, stat, stat,
                  pl.BlockSpec((BN, 1), lambda i, c: (i, 0)),
                  pl.BlockSpec((128, H), lambda i, c: (c, 0))],
        out_specs=[pl.BlockSpec((BN, H), lambda i, c: (i, 0)),
                   pl.BlockSpec((BN, 128), lambda i, c: (c * GN + i, 0))],
        out_shape=[jax.ShapeDtypeStruct((N, H), _f32),
                   jax.ShapeDtypeStruct((4 * N, 128), _f32)],
    )(y, cs, vs, ms, nw, nb, dis, Wn0)


def _t_epi3l3(y, cs, vs, ms, nw, nb, dis, W3):
    def body(y_ref, cs_ref, vs_ref, ms_ref, nw_ref, nb_ref, dis_ref, w_ref,
             p_ref, u_ref):
        c = pl.program_id(1)
        g = _gnorm_block(y_ref[...], cs_ref[...], vs_ref[...], ms_ref[...],
                         nw_ref[...], nb_ref[...])
        for k in range(4):
            part = jnp.dot(g, w_ref[k], preferred_element_type=_f32)

            @pl.when(c == 0)
            def _(part=part, k=k):
                p_ref[k] = part

            @pl.when(c != 0)
            def _(part=part, k=k):
                p_ref[k] = p_ref[k] + part

        u_ref[...] = jnp.concatenate(
            [p_ref[3] * dis_ref[...], jnp.zeros((BN, 128 - C), _f32)], axis=1)

    stat = pl.BlockSpec((1, 128), lambda i, c: (0, c))
    return pl.pallas_call(
        body,
        grid=(GN, 4),
        in_specs=[pl.BlockSpec((BN, 128), lambda i, c: (i, c)),
                  stat, stat, stat, stat, stat,
                  pl.BlockSpec((BN, 1), lambda i, c: (i, 0)),
                  pl.BlockSpec((4, 128, C), lambda i, c: (0, c, 0))],
        out_specs=[pl.BlockSpec((4, BN, C), lambda i, c: (0, i, 0)),
                   pl.BlockSpec((BN, 128), lambda i, c: (i, 0))],
        out_shape=[jax.ShapeDtypeStruct((4, N, C), _f32),
                   jax.ShapeDtypeStruct((N, 128), _f32)],
    )(y, cs, vs, ms, nw, nb, dis, W3)


def _t_l3hop(s2, p, dis, k):
    def body(sa_ref, sb_ref, p_ref, dis_ref, u_ref):
        disb = dis_ref[...]
        q = disb * (sa_ref[...] + sb_ref[...])[:, :C] + p_ref[0]
        u_ref[...] = jnp.concatenate(
            [disb * q, jnp.zeros((BN, 128 - C), _f32)], axis=1)

    return pl.pallas_call(
        body,
        grid=(GN,),
        in_specs=[pl.BlockSpec((BN, 128), lambda i: (i, 0)),
                  pl.BlockSpec((BN, 128), lambda i: (GN + i, 0)),
                  pl.BlockSpec((1, BN, C), lambda i: (3 - k, i, 0)),
                  pl.BlockSpec((BN, 1), lambda i: (i, 0))],
        out_specs=pl.BlockSpec((BN, 128), lambda i: (i, 0)),
        out_shape=jax.ShapeDtypeStruct((N, 128), _f32),
    )(s2, s2, p, dis)


def _t_l3final(s2, p, dis, b3):
    def body(sa_ref, sb_ref, p_ref, dis_ref, b_ref, out_ref):
        out_ref[...] = (dis_ref[...] * (sa_ref[...] + sb_ref[...])[:, :C]
                        + p_ref[0] + b_ref[...])

    return pl.pallas_call(
        body,
        grid=(GN,),
        in_specs=[pl.BlockSpec((BN, 128), lambda i: (i, 0)),
                  pl.BlockSpec((BN, 128), lambda i: (GN + i, 0)),
                  pl.BlockSpec((1, BN, C), lambda i: (0, i, 0)),
                  pl.BlockSpec((BN, 1), lambda i: (i, 0)),
                  pl.BlockSpec((1, C), lambda i: (0, 0))],
        out_specs=pl.BlockSpec((BN, C), lambda i: (i, 0)),
        out_shape=jax.ShapeDtypeStruct((N, C), _f32),
    )(s2, s2, p, dis, b3)


def kernel(x, weight, W1, b1, W2, b2, W3, b3, n1_w, n1_b, n1_ms, n2_w, n2_b,
           n2_ms, edge_index):
    row, col = edge_index[0], edge_index[1]
    padn = EP - E
    padidx = jnp.arange(padn, dtype=_i32) % N
    rowp = jnp.concatenate([row, padidx]).reshape(RPAD, EB)
    colp = jnp.concatenate([col, padidx]).reshape(RPAD, EB)
    wp = jnp.concatenate([weight, jnp.zeros((padn,), _f32)]).reshape(RPAD, EB)

    wflat = wp.reshape(EP)
    deg2 = _deg_call(colp, wp)
    dis = _t_dis(deg2[:N, None], deg2[N:, None])

    b1r, b2r, b3r = b1[None, :], b2[None, :], b3[None, :]
    ms1, ms2 = n1_ms[None, :], n2_ms[None, :]

    acc, u = _t_l1start(x, dis, W1[0])
    for k in range(1, 4):
        s = _hop_es(u, rowp, colp, wflat)
        acc, u = _t_hopacc_es(s, dis, W1[k], acc, last=(k == 3))
    y, cs = _t_epi1(acc, b1r)
    vs = _t_epi2(y, cs, ms1)
    acc, u = _t_epi3(y, cs, vs, ms1, n1_w[None, :], n1_b[None, :], dis, W2[0])

    for k in range(1, 4):
        s = _hop_l2(u, rowp, colp, wflat)
        acc, u = _t_hopacc(s, dis, W2[k], acc, ncT=4, Wc=128, last=(k == 3))
    y, cs = _t_epi1(acc, b2r)
    vs = _t_epi2(y, cs, ms2)
    p, u = _t_epi3l3(y, cs, vs, ms2, n2_w[None, :], n2_b[None, :], dis, W3)

    out = None
    for k in range(1, 4):
        s2 = _hop_es16(u, rowp, colp, wflat)
        if k < 3:
            u = _t_l3hop(s2, p, dis, k)
        else:
            out = _t_l3final(s2, p, dis, b3r)
    return out


# speedup vs baseline: 8.6189x; 1.0048x over previous
import functools

import jax
import jax.numpy as jnp
from jax import lax
from jax.experimental import pallas as pl
from jax.experimental.pallas import tpu as pltpu
from jax.experimental.pallas import tpu_sc as plsc

N = 10000
E = 320000
DIN = 128
H = 512
C = 16

NC = 2
NS = 16
EB = 128
RPAD = 2560
EP = RPAD * EB
BN = 1000
GN = N // BN
TROW = 640
LROW = N - 15 * TROW

_mesh = plsc.VectorSubcoreMesh(core_axis_name="c", subcore_axis_name="s",
                               num_cores=NC, num_subcores=NS)

_f32 = jnp.float32
_i32 = jnp.int32


def _make_deg_kernel():
    rpt = RPAD // (NC * NS)

    @functools.partial(
        pl.kernel,
        out_type=jax.ShapeDtypeStruct((NC * N,), _f32),
        mesh=_mesh,
        scratch_types=[
            pltpu.VMEM((rpt, EB), _i32),
            pltpu.VMEM((rpt, EB), _f32),
            pltpu.VMEM_SHARED((N,), _f32),
            pltpu.VMEM((1000,), _f32),
        ],
    )
    def deg_kernel(col_hbm, w_hbm, out_hbm, col_v, w_v, acc_sh, zbuf):
        cid = lax.axis_index("c")
        sid = lax.axis_index("s")
        jbase = (cid * NS + sid) * rpt
        pltpu.sync_copy(col_hbm.at[pl.ds(jbase, rpt)], col_v)
        pltpu.sync_copy(w_hbm.at[pl.ds(jbase, rpt)], w_v)

        @pl.when(sid == 0)
        def _():
            @pl.loop(0, 1000 // 16)
            def _(i):
                zbuf[pl.ds(i * 16, 16)] = jnp.zeros((16,), _f32)
            for z in range(N // 1000):
                pltpu.sync_copy(zbuf, acc_sh.at[pl.ds(z * 1000, 1000)])

        plsc.subcore_barrier()

        @pl.loop(0, rpt)
        def _(j):
            pltpu.sync_copy(w_v.at[j], acc_sh.at[col_v.at[j]], add=True)

        plsc.subcore_barrier()

        @pl.when(sid == 0)
        def _():
            for z in range(N // 1000):
                pltpu.sync_copy(acc_sh.at[pl.ds(z * 1000, 1000)], zbuf)
                pltpu.sync_copy(zbuf, out_hbm.at[pl.ds(cid * N + z * 1000, 1000)])

    return deg_kernel


def _make_hop_kernel(ncT, Wc, cpsc, edge_split, scale_cols=None):
    rpt = RPAD // (NC * NS) if edge_split else RPAD // NS
    out_rows = 2 * N if edge_split else ncT * N
    SB = 16
    ZR = 16
    OR = 80
    UE = 4
    if scale_cols is None:
        scale_cols = Wc

    scratch = [
        pltpu.VMEM((SB, EB), _i32),
        pltpu.VMEM((SB, EB), _i32),
        pltpu.VMEM((SB * EB,), _f32),
        pltpu.VMEM((EB, Wc), _f32),
        pltpu.VMEM((EB, Wc), _f32),
        pltpu.VMEM_SHARED((N, Wc), _f32),
        pltpu.VMEM((ZR, Wc), _f32),
        pltpu.SemaphoreType.DMA,
        pltpu.SemaphoreType.DMA,
        pltpu.SemaphoreType.DMA,
        pltpu.SemaphoreType.DMA,
    ]

    @functools.partial(
        pl.kernel,
        out_type=jax.ShapeDtypeStruct((out_rows, Wc), _f32),
        mesh=_mesh,
        scratch_types=scratch,
        compiler_params=pltpu.CompilerParams(needs_layout_passes=False),
    )
    def hop_kernel(u_hbm, row_hbm, col_hbm, w_hbm, s_hbm, row_v, col_v, w_v,
                   g0, g1, acc_sh, zbuf, sg0, sg1, ss0, ss1):
        cid = lax.axis_index("c")
        sid = lax.axis_index("s")
        jbase = ((cid * NS + sid) if edge_split else sid) * rpt
        gs = [(g0, sg0, ss0), (g1, sg1, ss1)]

        for r in range(ZR):
            for f in range(Wc // 16):
                zbuf[r, pl.ds(f * 16, 16)] = jnp.zeros((16,), _f32)

        def scale(g, j):
            @pl.loop(0, EB // UE)
            def _(eg):
                e0 = eg * UE
                for q in range(UE):
                    e = e0 + q
                    wb = plsc.load_gather(
                        w_v, [jnp.full((16,), j * EB + e, _i32)])
                    for f in range(scale_cols // 16):
                        v = g[e, pl.ds(f * 16, 16)]
                        g[e, pl.ds(f * 16, 16)] = v * wb

        for ci in range(cpsc):
            if ncT > 1:
                off = (cid * cpsc + ci) * N
            else:
                off = 0

            @pl.when(sid < 15)
            def _():
                for z in range(TROW // ZR):
                    pltpu.sync_copy(
                        zbuf, acc_sh.at[pl.ds(sid * TROW + z * ZR, ZR)])

            @pl.when(sid == 15)
            def _():
                for z in range(LROW // ZR):
                    pltpu.sync_copy(
                        zbuf, acc_sh.at[pl.ds(15 * TROW + z * ZR, ZR)])

            plsc.subcore_barrier()

            @pl.loop(0, rpt // SB)
            def _(b):
                jb = jbase + b * SB
                pltpu.sync_copy(row_hbm.at[pl.ds(jb, SB)], row_v)
                pltpu.sync_copy(col_hbm.at[pl.ds(jb, SB)], col_v)
                pltpu.sync_copy(w_hbm.at[pl.ds(jb * EB, SB * EB)], w_v)
                if ncT > 1:
                    offv = jnp.full((16,), off, _i32)

                    @pl.loop(0, SB)
                    def _(r):
                        for f in range(EB // 16):
                            row_v[r, pl.ds(f * 16, 16)] = (
                                row_v[r, pl.ds(f * 16, 16)] + offv)

                gdesc = [None, None]
                sdesc = [None, None]
                gdesc[0] = pltpu.async_copy(u_hbm.at[row_v.at[0]], g0, sg0)
                for j in range(SB):
                    cg, csg, css = gs[j % 2]
                    if j + 1 < SB:
                        ng, nsg, _ = gs[(j + 1) % 2]
                        if sdesc[(j + 1) % 2] is not None:
                            sdesc[(j + 1) % 2].wait()
                        gdesc[(j + 1) % 2] = pltpu.async_copy(
                            u_hbm.at[row_v.at[j + 1]], ng, nsg)
                    gdesc[j % 2].wait()
                    scale(cg, j)
                    sdesc[j % 2] = pltpu.async_copy(
                        cg, acc_sh.at[col_v.at[j]], css, add=True)
                sdesc[0].wait()
                sdesc[1].wait()

            plsc.subcore_barrier()
            base = cid * N if edge_split else off
            gb = g0.at[pl.ds(0, OR)]

            @pl.when(sid < 15)
            def _():
                for z in range(TROW // OR):
                    r0 = sid * TROW + z * OR
                    pltpu.sync_copy(acc_sh.at[pl.ds(r0, OR)], gb)
                    pltpu.sync_copy(gb, s_hbm.at[pl.ds(base + r0, OR)])

            @pl.when(sid == 15)
            def _():
                for z in range(LROW // OR):
                    r0 = 15 * TROW + z * OR
                    pltpu.sync_copy(acc_sh.at[pl.ds(r0, OR)], gb)
                    pltpu.sync_copy(gb, s_hbm.at[pl.ds(base + r0, OR)])

            if ci + 1 < cpsc:
                plsc.subcore_barrier()

    return hop_kernel


_deg_call = _make_deg_kernel()
_hop_es = _make_hop_kernel(ncT=1, Wc=128, cpsc=1, edge_split=True)
_hop_l2 = _make_hop_kernel(ncT=4, Wc=128, cpsc=2, edge_split=False)
_hop_es16 = _make_hop_kernel(ncT=1, Wc=128, cpsc=1, edge_split=True,
                             scale_cols=16)


def _t_dis(degA, degB):
    def body(a_ref, b_ref, dis_ref):
        d = a_ref[...] + b_ref[...]
        dis_ref[...] = jnp.where(d > 0, lax.rsqrt(jnp.where(d > 0, d, 1.0)), 0.0)

    return pl.pallas_call(
        body,
        grid=(GN,),
        in_specs=[pl.BlockSpec((BN, 1), lambda i: (i, 0)),
                  pl.BlockSpec((BN, 1), lambda i: (i, 0))],
        out_specs=pl.BlockSpec((BN, 1), lambda i: (i, 0)),
        out_shape=jax.ShapeDtypeStruct((N, 1), _f32),
    )(degA, degB)


def _t_l1start(x, dis, W0):
    def body(x_ref, dis_ref, w_ref, acc_ref, u_ref):
        xb = x_ref[...]
        u_ref[...] = xb * dis_ref[...]
        acc_ref[...] = jnp.dot(xb, w_ref[...], preferred_element_type=_f32)

    return pl.pallas_call(
        body,
        grid=(GN,),
        in_specs=[pl.BlockSpec((BN, DIN), lambda i: (i, 0)),
                  pl.BlockSpec((BN, 1), lambda i: (i, 0)),
                  pl.BlockSpec((DIN, H), lambda i: (0, 0))],
        out_specs=[pl.BlockSpec((BN, H), lambda i: (i, 0)),
                   pl.BlockSpec((BN, DIN), lambda i: (i, 0))],
        out_shape=[jax.ShapeDtypeStruct((N, H), _f32),
                   jax.ShapeDtypeStruct((N, DIN), _f32)],
    )(x, dis, W0)


def _t_hopacc_es(s2, dis, Wk, acc_in, last):
    def body(sa_ref, sb_ref, dis_ref, w_ref, acc_in_ref, acc_ref, *maybe_u):
        disb = dis_ref[...]
        hb = (sa_ref[...] + sb_ref[...]) * disb
        if maybe_u:
            maybe_u[0][...] = hb * disb
        acc_ref[...] = acc_in_ref[...] + jnp.dot(
            hb, w_ref[...], preferred_element_type=_f32)

    D = Wk.shape[0]
    out_specs = [pl.BlockSpec((BN, H), lambda i: (i, 0))]
    out_shape = [jax.ShapeDtypeStruct((N, H), _f32)]
    if not last:
        out_specs.append(pl.BlockSpec((BN, D), lambda i: (i, 0)))
        out_shape.append(jax.ShapeDtypeStruct((N, D), _f32))

    res = pl.pallas_call(
        body,
        grid=(GN,),
        in_specs=[pl.BlockSpec((BN, D), lambda i: (i, 0)),
                  pl.BlockSpec((BN, D), lambda i: (GN + i, 0)),
                  pl.BlockSpec((BN, 1), lambda i: (i, 0)),
                  pl.BlockSpec((D, H), lambda i: (0, 0)),
                  pl.BlockSpec((BN, H), lambda i: (i, 0))],
        out_specs=out_specs,
        out_shape=out_shape,
    )(s2, s2, dis, Wk, acc_in)
    return res if not last else (res[0], None)


def _t_hopacc(s, dis, Wk, acc_in, ncT, Wc, last):
    def body(s_ref, dis_ref, w_ref, acc_in_ref, acc_ref, *maybe_u):
        c = pl.program_id(1)
        disb = dis_ref[...]
        hb = s_ref[...] * disb
        if maybe_u:
            maybe_u[0][...] = hb * disb
        part = jnp.dot(hb, w_ref[...], preferred_element_type=_f32)

        @pl.when(c == 0)
        def _():
            acc_ref[...] = acc_in_ref[...] + part

        @pl.when(c != 0)
        def _():
            acc_ref[...] = acc_ref[...] + part

    out_specs = [pl.BlockSpec((BN, H), lambda i, c: (i, 0))]
    out_shape = [jax.ShapeDtypeStruct((N, H), _f32)]
    if not last:
        out_specs.append(pl.BlockSpec((BN, Wc), lambda i, c: (c * GN + i, 0)))
        out_shape.append(jax.ShapeDtypeStruct((ncT * N, Wc), _f32))

    res = pl.pallas_call(
        body,
        grid=(GN, ncT),
        in_specs=[pl.BlockSpec((BN, Wc), lambda i, c: (c * GN + i, 0)),
                  pl.BlockSpec((BN, 1), lambda i, c: (i, 0)),
                  pl.BlockSpec((Wc, H), lambda i, c: (c, 0)),
                  pl.BlockSpec((BN, H), lambda i, c: (i, 0))],
        out_specs=out_specs,
        out_shape=out_shape,
    )(s, dis, Wk, acc_in)
    return res if not last else (res[0], None)


def _t_epi12(acc, b):
    def body(acc_ref, b_ref, y_ref, cs_ref, cs2_ref):
        i = pl.program_id(0)
        t = acc_ref[...] + b_ref[...]
        y = jnp.where(t > 0, t, jnp.exp(jnp.minimum(t, 0.0)) - 1.0)
        y_ref[...] = y
        s = jnp.sum(y, axis=0, keepdims=True)
        s2 = jnp.sum(y * y, axis=0, keepdims=True)

        @pl.when(i == 0)
        def _():
            cs_ref[...] = s
            cs2_ref[...] = s2

        @pl.when(i != 0)
        def _():
            cs_ref[...] = cs_ref[...] + s
            cs2_ref[...] = cs2_ref[...] + s2

    return pl.pallas_call(
        body,
        grid=(GN,),
        in_specs=[pl.BlockSpec((BN, H), lambda i: (i, 0)),
                  pl.BlockSpec((1, H), lambda i: (0, 0))],
        out_specs=[pl.BlockSpec((BN, H), lambda i: (i, 0)),
                   pl.BlockSpec((1, H), lambda i: (0, 0)),
                   pl.BlockSpec((1, H), lambda i: (0, 0))],
        out_shape=[jax.ShapeDtypeStruct((N, H), _f32),
                   jax.ShapeDtypeStruct((1, H), _f32),
                   jax.ShapeDtypeStruct((1, H), _f32)],
    )(acc, b)


def _gnorm_block(y, cs, cs2, ms, nw, nb):
    mean = cs * (1.0 / N)
    q = cs2 * (1.0 / N)
    d = y - ms * mean
    var = q - (2.0 * ms - ms * ms) * mean * mean
    return nw * d * lax.rsqrt(var + 1e-5) + nb


def _t_epi3(y, cs, vs, ms, nw, nb, dis, Wn0):
    def body(y_ref, cs_ref, vs_ref, ms_ref, nw_ref, nb_ref, dis_ref, w_ref,
             acc_ref, u_ref):
        c = pl.program_id(1)
        g = _gnorm_block(y_ref[...], cs_ref[...], vs_ref[...], ms_ref[...],
                         nw_ref[...], nb_ref[...])
        u_ref[...] = g * dis_ref[...]
        part = jnp.dot(g, w_ref[...], preferred_element_type=_f32)

        @pl.when(c == 0)
        def _():
            acc_ref[...] = part

        @pl.when(c != 0)
        def _():
            acc_ref[...] = acc_ref[...] + part

    stat = pl.BlockSpec((1, 128), lambda i, c: (0, c))
    return pl.pallas_call(
        body,
        grid=(GN, 4),
        in_specs=[pl.BlockSpec((BN, 128), lambda i, c: (i, c)),
                  stat, stat, stat, stat, stat,
                  pl.BlockSpec((BN, 1), lambda i, c: (i, 0)),
                  pl.BlockSpec((128, H), lambda i, c: (c, 0))],
        out_specs=[pl.BlockSpec((BN, H), lambda i, c: (i, 0)),
                   pl.BlockSpec((BN, 128), lambda i, c: (c * GN + i, 0))],
        out_shape=[jax.ShapeDtypeStruct((N, H), _f32),
                   jax.ShapeDtypeStruct((4 * N, 128), _f32)],
    )(y, cs, vs, ms, nw, nb, dis, Wn0)


def _t_epi3l3(y, cs, vs, ms, nw, nb, dis, W3):
    def body(y_ref, cs_ref, vs_ref, ms_ref, nw_ref, nb_ref, dis_ref, w_ref,
             p_ref, u_ref):
        c = pl.program_id(1)
        g = _gnorm_block(y_ref[...], cs_ref[...], vs_ref[...], ms_ref[...],
                         nw_ref[...], nb_ref[...])
        for k in range(4):
            part = jnp.dot(g, w_ref[k], preferred_element_type=_f32)

            @pl.when(c == 0)
            def _(part=part, k=k):
                p_ref[k] = part

            @pl.when(c != 0)
            def _(part=part, k=k):
                p_ref[k] = p_ref[k] + part

        u_ref[...] = jnp.concatenate(
            [p_ref[3] * dis_ref[...], jnp.zeros((BN, 128 - C), _f32)], axis=1)

    stat = pl.BlockSpec((1, 128), lambda i, c: (0, c))
    return pl.pallas_call(
        body,
        grid=(GN, 4),
        in_specs=[pl.BlockSpec((BN, 128), lambda i, c: (i, c)),
                  stat, stat, stat, stat, stat,
                  pl.BlockSpec((BN, 1), lambda i, c: (i, 0)),
                  pl.BlockSpec((4, 128, C), lambda i, c: (0, c, 0))],
        out_specs=[pl.BlockSpec((4, BN, C), lambda i, c: (0, i, 0)),
                   pl.BlockSpec((BN, 128), lambda i, c: (i, 0))],
        out_shape=[jax.ShapeDtypeStruct((4, N, C), _f32),
                   jax.ShapeDtypeStruct((N, 128), _f32)],
    )(y, cs, vs, ms, nw, nb, dis, W3)


def _t_l3hop(s2, p, dis, k):
    def body(sa_ref, sb_ref, p_ref, dis_ref, u_ref):
        disb = dis_ref[...]
        q = disb * (sa_ref[...] + sb_ref[...])[:, :C] + p_ref[0]
        u_ref[...] = jnp.concatenate(
            [disb * q, jnp.zeros((BN, 128 - C), _f32)], axis=1)

    return pl.pallas_call(
        body,
        grid=(GN,),
        in_specs=[pl.BlockSpec((BN, 128), lambda i: (i, 0)),
                  pl.BlockSpec((BN, 128), lambda i: (GN + i, 0)),
                  pl.BlockSpec((1, BN, C), lambda i: (3 - k, i, 0)),
                  pl.BlockSpec((BN, 1), lambda i: (i, 0))],
        out_specs=pl.BlockSpec((BN, 128), lambda i: (i, 0)),
        out_shape=jax.ShapeDtypeStruct((N, 128), _f32),
    )(s2, s2, p, dis)


def _t_l3final(s2, p, dis, b3):
    def body(sa_ref, sb_ref, p_ref, dis_ref, b_ref, out_ref):
        out_ref[...] = (dis_ref[...] * (sa_ref[...] + sb_ref[...])[:, :C]
                        + p_ref[0] + b_ref[...])

    return pl.pallas_call(
        body,
        grid=(GN,),
        in_specs=[pl.BlockSpec((BN, 128), lambda i: (i, 0)),
                  pl.BlockSpec((BN, 128), lambda i: (GN + i, 0)),
                  pl.BlockSpec((1, BN, C), lambda i: (0, i, 0)),
                  pl.BlockSpec((BN, 1), lambda i: (i, 0)),
                  pl.BlockSpec((1, C), lambda i: (0, 0))],
        out_specs=pl.BlockSpec((BN, C), lambda i: (i, 0)),
        out_shape=jax.ShapeDtypeStruct((N, C), _f32),
    )(s2, s2, p, dis, b3)


def kernel(x, weight, W1, b1, W2, b2, W3, b3, n1_w, n1_b, n1_ms, n2_w, n2_b,
           n2_ms, edge_index):
    row, col = edge_index[0], edge_index[1]
    padn = EP - E
    padidx = jnp.arange(padn, dtype=_i32) % N
    rowp = jnp.concatenate([row, padidx]).reshape(RPAD, EB)
    colp = jnp.concatenate([col, padidx]).reshape(RPAD, EB)
    wp = jnp.concatenate([weight, jnp.zeros((padn,), _f32)]).reshape(RPAD, EB)

    wflat = wp.reshape(EP)
    deg2 = _deg_call(colp, wp)
    dis = _t_dis(deg2[:N, None], deg2[N:, None])

    b1r, b2r, b3r = b1[None, :], b2[None, :], b3[None, :]
    ms1, ms2 = n1_ms[None, :], n2_ms[None, :]

    acc, u = _t_l1start(x, dis, W1[0])
    for k in range(1, 4):
        s = _hop_es(u, rowp, colp, wflat)
        acc, u = _t_hopacc_es(s, dis, W1[k], acc, last=(k == 3))
    y, cs, cs2 = _t_epi12(acc, b1r)
    acc, u = _t_epi3(y, cs, cs2, ms1, n1_w[None, :], n1_b[None, :], dis, W2[0])

    for k in range(1, 4):
        s = _hop_l2(u, rowp, colp, wflat)
        acc, u = _t_hopacc(s, dis, W2[k], acc, ncT=4, Wc=128, last=(k == 3))
    y, cs, cs2 = _t_epi12(acc, b2r)
    p, u = _t_epi3l3(y, cs, cs2, ms2, n2_w[None, :], n2_b[None, :], dis, W3)

    out = None
    for k in range(1, 4):
        s2 = _hop_es16(u, rowp, colp, wflat)
        if k < 3:
            u = _t_l3hop(s2, p, dis, k)
        else:
            out = _t_l3final(s2, p, dis, b3r)
    return out
